# baseline (speedup 1.0000x reference)
import functools

import jax
import jax.numpy as jnp
import numpy as np
from jax import lax
from jax.experimental import pallas as pl
from jax.experimental.pallas import tpu as pltpu
from jax.experimental.pallas import tpu_sc as plsc

_NUM_EDGES = 320000
_NUM_NODES = 10000
_NUM_ROWS = 128
_NUM_PIV = 100
_ROW_STRIDE = _NUM_EDGES + _NUM_NODES

_FLAT_IDX = (
    np.arange(_NUM_ROWS, dtype=np.int64)[:, None] * _ROW_STRIDE
    + _NUM_EDGES
    + 100 * np.arange(_NUM_PIV, dtype=np.int64)[None, :]
).astype(np.int32)

_NC = 2
_NS = 16
_NW = _NC * _NS
_ROWS_PER_W = _NUM_ROWS // _NW


def _sc_gather(x_flat, idx):
    mesh = plsc.VectorSubcoreMesh(core_axis_name="c", subcore_axis_name="s")

    @functools.partial(
        pl.kernel,
        mesh=mesh,
        out_type=jax.ShapeDtypeStruct((_NUM_ROWS, _NUM_PIV), jnp.float32),
        scratch_types=[
            pltpu.VMEM((_ROWS_PER_W, _NUM_PIV), jnp.int32),
            pltpu.VMEM((_ROWS_PER_W, _NUM_PIV), jnp.float32),
            pltpu.SemaphoreType.DMA,
        ],
    )
    def run(x_hbm, idx_hbm, out_hbm, idx_v, rows_v, sem):
        wid = lax.axis_index("s") * _NC + lax.axis_index("c")
        r0 = wid * _ROWS_PER_W
        pltpu.sync_copy(idx_hbm.at[pl.ds(r0, _ROWS_PER_W), :], idx_v)
        copies = [
            pltpu.async_copy(x_hbm.at[idx_v.at[j]], rows_v.at[j], sem)
            for j in range(_ROWS_PER_W)
        ]
        for c in copies:
            c.wait()
        pltpu.sync_copy(rows_v, out_hbm.at[pl.ds(r0, _ROWS_PER_W), :])

    return run(x_flat, idx)


def kernel(x):
    x_flat = x.reshape(-1)
    idx = jnp.asarray(_FLAT_IDX)
    return _sc_gather(x_flat, idx)


# speedup vs baseline: 13.4568x; 13.4568x over previous
import functools

import jax
import jax.numpy as jnp
from jax import lax
from jax.experimental import pallas as pl
from jax.experimental.pallas import tpu as pltpu
from jax.experimental.pallas import tpu_sc as plsc

_NUM_EDGES = 320000
_NUM_NODES = 10000
_NUM_ROWS = 128
_NUM_PIV = 100
_PIV_STRIDE = 100

_NC = 2
_NS = 16
_NW = _NC * _NS
_ROWS_PER_W = _NUM_ROWS // _NW
_L = 16
_CHUNKS = 7
_OUT_PER_W = _ROWS_PER_W * _NUM_PIV


def _sc_gather(x):
    mesh = plsc.VectorSubcoreMesh(core_axis_name="c", subcore_axis_name="s")

    @functools.partial(
        pl.kernel,
        mesh=mesh,
        compiler_params=pltpu.CompilerParams(needs_layout_passes=False),
        out_type=jax.ShapeDtypeStruct((_NUM_ROWS * _NUM_PIV,), jnp.float32),
        scratch_types=(
            [pltpu.VMEM((_NUM_NODES,), jnp.float32) for _ in range(_ROWS_PER_W)]
            + [pltpu.VMEM((_OUT_PER_W + _L,), jnp.float32)]
            + [pltpu.SemaphoreType.DMA]
        ),
    )
    def run(x_hbm, out_hbm, *rest):
        win_v = rest[:_ROWS_PER_W]
        out_v = rest[_ROWS_PER_W]
        sem = rest[_ROWS_PER_W + 1]
        wid = lax.axis_index("s") * _NC + lax.axis_index("c")
        r0 = wid * _ROWS_PER_W
        copies = [
            pltpu.async_copy(
                x_hbm.at[r0 + j, pl.ds(_NUM_EDGES, _NUM_NODES)],
                win_v[j],
                sem,
            )
            for j in range(_ROWS_PER_W)
        ]
        for c in copies:
            c.wait()
        for j in range(_ROWS_PER_W):
            for t in range(_CHUNKS):
                idx = (lax.iota(jnp.int32, _L) + (t * _L)) * _PIV_STRIDE
                idx = jnp.minimum(idx, _NUM_NODES - _PIV_STRIDE)
                vals = plsc.load_gather(win_v[j], [idx])
                out_v[pl.ds(j * _NUM_PIV + t * _L, _L)] = vals
        pltpu.sync_copy(
            out_v.at[pl.ds(0, _OUT_PER_W)],
            out_hbm.at[pl.ds(wid * _OUT_PER_W, _OUT_PER_W)],
        )

    return run(x)


def kernel(x):
    return _sc_gather(x).reshape(_NUM_ROWS, _NUM_PIV)
